# baseline (speedup 1.0000x reference)
import functools

import jax
import jax.numpy as jnp
from jax import lax
from jax.experimental import pallas as pl
from jax.experimental.pallas import tpu as pltpu
from jax.experimental.pallas import tpu_sc as plsc

N = 10000
E = 160000
D = 256
H = 128
NC = 2
NS = 16
STRIPE = 624
TAIL = N - NS * STRIPE
TAIL_OFF = NS * STRIPE
EPT_HOP = E // NS
EPT_DEG = E // (NC * NS)
CH = 80
CH_D = 40

_MESH = plsc.VectorSubcoreMesh(
    core_axis_name="c", subcore_axis_name="s", num_cores=NC, num_subcores=NS
)


def _stripe_copy(src, dst, s):
    pltpu.sync_copy(
        src.at[pl.ds(s * STRIPE, STRIPE)], dst.at[pl.ds(s * STRIPE, STRIPE)]
    )
    @pl.when(s == 0)
    def _():
        pltpu.sync_copy(
            src.at[pl.ds(TAIL_OFF, TAIL)], dst.at[pl.ds(TAIL_OFF, TAIL)]
        )


@functools.partial(
    pl.kernel,
    out_type=jax.ShapeDtypeStruct((NC * N,), jnp.float32),
    mesh=_MESH,
    scratch_types=[
        pltpu.VMEM((EPT_DEG,), jnp.int32),
        pltpu.VMEM((CH_D,), jnp.float32),
        pltpu.VMEM((STRIPE,), jnp.float32),
        pltpu.VMEM_SHARED((N,), jnp.float32),
        pltpu.SemaphoreType.DMA,
    ],
)
def _deg_kernel(row_hbm, zeros_hbm, ones_hbm, out_hbm, idx_v, ones_v,
                stg_v, acc, sem_d):
    c = lax.axis_index("c")
    s = lax.axis_index("s")
    pltpu.sync_copy(zeros_hbm.at[pl.ds(0, STRIPE)], stg_v)
    pltpu.sync_copy(stg_v, acc.at[pl.ds(s * STRIPE, STRIPE)])
    @pl.when(s == 0)
    def _():
        pltpu.sync_copy(stg_v.at[pl.ds(0, TAIL)], acc.at[pl.ds(TAIL_OFF, TAIL)])
    pltpu.sync_copy(ones_hbm, ones_v)
    pltpu.sync_copy(row_hbm.at[pl.ds((c * NS + s) * EPT_DEG, EPT_DEG)], idx_v)
    plsc.subcore_barrier()

    def body(k, _):
        ia = idx_v.at[pl.ds(2 * k * CH_D, CH_D)]
        ib = idx_v.at[pl.ds((2 * k + 1) * CH_D, CH_D)]
        pltpu.async_copy(ones_v, acc.at[ia], sem_d, add=True)
        pltpu.async_copy(ones_v, acc.at[ib], sem_d, add=True)
        pltpu.make_async_copy(ones_v, acc.at[ia], sem_d).wait()
        pltpu.make_async_copy(ones_v, acc.at[ib], sem_d).wait()
        return 0

    nch_d = EPT_DEG // CH_D
    lax.fori_loop(0, nch_d // 2, body, 0)
    pltpu.sync_copy(
        ones_v, acc.at[idx_v.at[pl.ds((nch_d - 1) * CH_D, CH_D)]], add=True
    )
    plsc.subcore_barrier()
    pltpu.sync_copy(acc.at[pl.ds(s * STRIPE, STRIPE)], stg_v)
    pltpu.sync_copy(stg_v, out_hbm.at[pl.ds(c * N + s * STRIPE, STRIPE)])
    @pl.when(s == 0)
    def _():
        pltpu.sync_copy(acc.at[pl.ds(TAIL_OFF, TAIL)], stg_v.at[pl.ds(0, TAIL)])
        pltpu.sync_copy(
            stg_v.at[pl.ds(0, TAIL)], out_hbm.at[pl.ds(c * N + TAIL_OFF, TAIL)]
        )


@functools.partial(
    pl.kernel,
    out_type=jax.ShapeDtypeStruct((NC, N, H), jnp.float32),
    mesh=_MESH,
    scratch_types=[
        pltpu.VMEM((EPT_HOP,), jnp.int32),
        pltpu.VMEM((EPT_HOP,), jnp.int32),
        pltpu.VMEM((CH, H), jnp.float32),
        pltpu.VMEM((CH, H), jnp.float32),
        pltpu.VMEM_SHARED((N, H), jnp.float32),
        pltpu.SemaphoreType.DMA,
        pltpu.SemaphoreType.DMA,
        pltpu.SemaphoreType.DMA,
        pltpu.SemaphoreType.DMA,
    ],
)
def _hop_kernel(y_hbm, col_hbm, row_hbm, zeros_hbm, out_hbm,
                col_v, row_v, buf_a, buf_b, acc, sem_a, sem_b, sem_sa, sem_sb):
    c = lax.axis_index("c")
    s = lax.axis_index("s")
    _stripe_copy(zeros_hbm, acc, s)
    y_half = y_hbm.at[c]
    nch = EPT_HOP // CH
    pltpu.sync_copy(col_hbm.at[pl.ds(s * EPT_HOP, EPT_HOP)], col_v)
    pltpu.sync_copy(row_hbm.at[pl.ds(s * EPT_HOP, EPT_HOP)], row_v)
    plsc.subcore_barrier()

    def cidx(k):
        return col_v.at[pl.ds(k * CH, CH)]

    def ridx(k):
        return row_v.at[pl.ds(k * CH, CH)]

    pltpu.async_copy(y_half.at[cidx(0)], buf_a, sem_a)
    pltpu.async_copy(y_half.at[cidx(1)], buf_b, sem_b)

    def pair(i, _):
        k = 2 * i
        pltpu.make_async_copy(y_half.at[cidx(k)], buf_a, sem_a).wait()
        pltpu.async_copy(buf_a, acc.at[ridx(k)], sem_sa, add=True)
        pltpu.make_async_copy(y_half.at[cidx(k + 1)], buf_b, sem_b).wait()
        pltpu.async_copy(buf_b, acc.at[ridx(k + 1)], sem_sb, add=True)
        pltpu.make_async_copy(buf_a, acc.at[ridx(k)], sem_sa).wait()
        pltpu.async_copy(y_half.at[cidx(k + 2)], buf_a, sem_a)
        pltpu.make_async_copy(buf_b, acc.at[ridx(k + 1)], sem_sb).wait()
        pltpu.async_copy(y_half.at[cidx(k + 3)], buf_b, sem_b)
        return 0

    lax.fori_loop(0, (nch - 3) // 2, pair, 0)
    pltpu.make_async_copy(y_half.at[cidx(nch - 3)], buf_a, sem_a).wait()
    pltpu.async_copy(buf_a, acc.at[ridx(nch - 3)], sem_sa, add=True)
    pltpu.make_async_copy(y_half.at[cidx(nch - 2)], buf_b, sem_b).wait()
    pltpu.async_copy(buf_b, acc.at[ridx(nch - 2)], sem_sb, add=True)
    pltpu.make_async_copy(buf_a, acc.at[ridx(nch - 3)], sem_sa).wait()
    pltpu.async_copy(y_half.at[cidx(nch - 1)], buf_a, sem_a)
    pltpu.make_async_copy(buf_b, acc.at[ridx(nch - 2)], sem_sb).wait()
    pltpu.make_async_copy(y_half.at[cidx(nch - 1)], buf_a, sem_a).wait()
    pltpu.sync_copy(buf_a, acc.at[ridx(nch - 1)], add=True)
    plsc.subcore_barrier()
    _stripe_copy(acc, out_hbm.at[c], s)


def _prep_body(x_ref, degp_ref, y0_ref, deg_ref):
    xh = x_ref[...]
    n = jnp.float32(N)
    mean = jnp.sum(xh, axis=0, keepdims=True) / n
    xc = xh - mean
    var = jnp.sum(xc * xc, axis=0, keepdims=True) / (n - 1.0)
    std = jnp.sqrt(var)
    std = jnp.where(std == 0.0, 1.0, std)
    deg = degp_ref[0] + degp_ref[1] + 1.0
    d = lax.rsqrt(deg)
    y0_ref[...] = (d * (xc / std))[None]
    deg_ref[...] = deg


def _mid_body(agg_ref, y_ref, deg_ref, out_ref):
    d2 = 1.0 / deg_ref[...]
    out_ref[...] = d2[None] * (agg_ref[...] + y_ref[...])


def _final_body(agg_ref, y_ref, deg_ref, out_ref):
    d = lax.rsqrt(deg_ref[...])
    out_ref[...] = d * (agg_ref[0] + y_ref[0])


_prep = pl.pallas_call(
    _prep_body,
    grid=(NC,),
    in_specs=[
        pl.BlockSpec((N, H), lambda c: (0, c)),
        pl.BlockSpec((NC, N, 1), lambda c: (0, 0, 0)),
    ],
    out_specs=[
        pl.BlockSpec((1, N, H), lambda c: (c, 0, 0)),
        pl.BlockSpec((N, 1), lambda c: (0, 0)),
    ],
    out_shape=[
        jax.ShapeDtypeStruct((NC, N, H), jnp.float32),
        jax.ShapeDtypeStruct((N, 1), jnp.float32),
    ],
)

_mid = pl.pallas_call(
    _mid_body,
    grid=(NC,),
    in_specs=[
        pl.BlockSpec((1, N, H), lambda c: (c, 0, 0)),
        pl.BlockSpec((1, N, H), lambda c: (c, 0, 0)),
        pl.BlockSpec((N, 1), lambda c: (0, 0)),
    ],
    out_specs=pl.BlockSpec((1, N, H), lambda c: (c, 0, 0)),
    out_shape=jax.ShapeDtypeStruct((NC, N, H), jnp.float32),
)

_final = pl.pallas_call(
    _final_body,
    grid=(NC,),
    in_specs=[
        pl.BlockSpec((1, N, H), lambda c: (c, 0, 0)),
        pl.BlockSpec((1, N, H), lambda c: (c, 0, 0)),
        pl.BlockSpec((N, 1), lambda c: (0, 0)),
    ],
    out_specs=pl.BlockSpec((N, H), lambda c: (0, c)),
    out_shape=jax.ShapeDtypeStruct((N, D), jnp.float32),
)


def kernel(x, edge_index):
    row = edge_index[0]
    col = edge_index[1]
    zeros_nh = jnp.zeros((N, H), jnp.float32)
    deg_parts = _deg_kernel(
        row, jnp.zeros((N,), jnp.float32), jnp.ones((CH_D,), jnp.float32)
    ).reshape(NC, N, 1)
    y0, deg = _prep(x, deg_parts)
    agg0 = _hop_kernel(y0, col, row, zeros_nh)
    y1 = _mid(agg0, y0, deg)
    agg1 = _hop_kernel(y1, col, row, zeros_nh)
    return _final(agg1, y1, deg)


# speedup vs baseline: 1.2523x; 1.2523x over previous
import functools

import jax
import jax.numpy as jnp
from jax import lax
from jax.experimental import pallas as pl
from jax.experimental.pallas import tpu as pltpu
from jax.experimental.pallas import tpu_sc as plsc

N = 10000
E = 160000
D = 256
H = 128
NC = 2
NS = 16
STRIPE = 624
TAIL = N - NS * STRIPE
TAIL_OFF = NS * STRIPE
EPT_HOP = E // NS
EPT_DEG = E // (NC * NS)
CH = 80
CH_D = 40

_MESH = plsc.VectorSubcoreMesh(
    core_axis_name="c", subcore_axis_name="s", num_cores=NC, num_subcores=NS
)


def _stripe_copy(src, dst, s):
    pltpu.sync_copy(
        src.at[pl.ds(s * STRIPE, STRIPE)], dst.at[pl.ds(s * STRIPE, STRIPE)]
    )
    @pl.when(s == 0)
    def _():
        pltpu.sync_copy(
            src.at[pl.ds(TAIL_OFF, TAIL)], dst.at[pl.ds(TAIL_OFF, TAIL)]
        )


@functools.partial(
    pl.kernel,
    out_type=jax.ShapeDtypeStruct((NC * N,), jnp.float32),
    mesh=_MESH,
    scratch_types=[
        pltpu.VMEM((EPT_DEG,), jnp.int32),
        pltpu.VMEM((CH_D,), jnp.float32),
        pltpu.VMEM((STRIPE,), jnp.float32),
        pltpu.VMEM_SHARED((N,), jnp.float32),
        pltpu.SemaphoreType.DMA,
    ],
)
def _deg_kernel(row_hbm, zeros_hbm, ones_hbm, out_hbm, idx_v, ones_v,
                stg_v, acc, sem_d):
    c = lax.axis_index("c")
    s = lax.axis_index("s")
    pltpu.sync_copy(zeros_hbm.at[pl.ds(0, STRIPE)], stg_v)
    pltpu.sync_copy(stg_v, acc.at[pl.ds(s * STRIPE, STRIPE)])
    @pl.when(s == 0)
    def _():
        pltpu.sync_copy(stg_v.at[pl.ds(0, TAIL)], acc.at[pl.ds(TAIL_OFF, TAIL)])
    pltpu.sync_copy(ones_hbm, ones_v)
    pltpu.sync_copy(row_hbm.at[pl.ds((c * NS + s) * EPT_DEG, EPT_DEG)], idx_v)
    plsc.subcore_barrier()

    def body(k, _):
        ia = idx_v.at[pl.ds(2 * k * CH_D, CH_D)]
        ib = idx_v.at[pl.ds((2 * k + 1) * CH_D, CH_D)]
        pltpu.async_copy(ones_v, acc.at[ia], sem_d, add=True)
        pltpu.async_copy(ones_v, acc.at[ib], sem_d, add=True)
        pltpu.make_async_copy(ones_v, acc.at[ia], sem_d).wait()
        pltpu.make_async_copy(ones_v, acc.at[ib], sem_d).wait()
        return 0

    nch_d = EPT_DEG // CH_D
    lax.fori_loop(0, nch_d // 2, body, 0)
    pltpu.sync_copy(
        ones_v, acc.at[idx_v.at[pl.ds((nch_d - 1) * CH_D, CH_D)]], add=True
    )
    plsc.subcore_barrier()
    pltpu.sync_copy(acc.at[pl.ds(s * STRIPE, STRIPE)], stg_v)
    pltpu.sync_copy(stg_v, out_hbm.at[pl.ds(c * N + s * STRIPE, STRIPE)])
    @pl.when(s == 0)
    def _():
        pltpu.sync_copy(acc.at[pl.ds(TAIL_OFF, TAIL)], stg_v.at[pl.ds(0, TAIL)])
        pltpu.sync_copy(
            stg_v.at[pl.ds(0, TAIL)], out_hbm.at[pl.ds(c * N + TAIL_OFF, TAIL)]
        )


@functools.partial(
    pl.kernel,
    out_type=jax.ShapeDtypeStruct((NC, N, H), jnp.float32),
    mesh=_MESH,
    scratch_types=[
        pltpu.VMEM((EPT_HOP,), jnp.int32),
        pltpu.VMEM((EPT_HOP,), jnp.int32),
        pltpu.VMEM((CH, H), jnp.float32),
        pltpu.VMEM((CH, H), jnp.float32),
        pltpu.VMEM_SHARED((N, H), jnp.float32),
        pltpu.SemaphoreType.DMA,
        pltpu.SemaphoreType.DMA,
    ],
)
def _hop_kernel(y_hbm, col_hbm, row_hbm, out_hbm,
                col_v, row_v, buf_a, buf_b, acc, sem_a, sem_b):
    c = lax.axis_index("c")
    s = lax.axis_index("s")
    def zrow(r, _):
        for j in range(H // 16):
            buf_a[r, pl.ds(j * 16, 16)] = jnp.zeros((16,), jnp.float32)
        return 0

    lax.fori_loop(0, CH, zrow, 0)
    for j in range(STRIPE // CH):
        pltpu.sync_copy(buf_a, acc.at[pl.ds(s * STRIPE + j * CH, CH)])
    pltpu.sync_copy(
        buf_a.at[pl.ds(0, STRIPE - (STRIPE // CH) * CH)],
        acc.at[pl.ds(s * STRIPE + (STRIPE // CH) * CH,
                     STRIPE - (STRIPE // CH) * CH)],
    )
    @pl.when(s == 0)
    def _():
        pltpu.sync_copy(buf_a.at[pl.ds(0, TAIL)], acc.at[pl.ds(TAIL_OFF, TAIL)])
    y_half = y_hbm.at[c]
    nch = EPT_HOP // CH
    pltpu.sync_copy(col_hbm.at[pl.ds(s * EPT_HOP, EPT_HOP)], col_v)
    pltpu.sync_copy(row_hbm.at[pl.ds(s * EPT_HOP, EPT_HOP)], row_v)
    plsc.subcore_barrier()

    def cidx(k):
        return col_v.at[pl.ds(k * CH, CH)]

    def ridx(k):
        return row_v.at[pl.ds(k * CH, CH)]

    pltpu.async_copy(y_half.at[cidx(0)], buf_a, sem_a)

    def pair(i, _):
        k = 2 * i
        pltpu.async_copy(y_half.at[cidx(k + 1)], buf_b, sem_b)
        pltpu.make_async_copy(y_half.at[cidx(k)], buf_a, sem_a).wait()
        pltpu.sync_copy(buf_a, acc.at[ridx(k)], add=True)
        pltpu.async_copy(y_half.at[cidx(k + 2)], buf_a, sem_a)
        pltpu.make_async_copy(y_half.at[cidx(k + 1)], buf_b, sem_b).wait()
        pltpu.sync_copy(buf_b, acc.at[ridx(k + 1)], add=True)
        return 0

    lax.fori_loop(0, (nch - 1) // 2, pair, 0)
    pltpu.make_async_copy(y_half.at[cidx(nch - 1)], buf_a, sem_a).wait()
    pltpu.sync_copy(buf_a, acc.at[ridx(nch - 1)], add=True)
    plsc.subcore_barrier()
    _stripe_copy(acc, out_hbm.at[c], s)


def _prep_body(x_ref, degp_ref, y0_ref, deg_ref):
    xh = x_ref[...]
    n = jnp.float32(N)
    mean = jnp.sum(xh, axis=0, keepdims=True) / n
    xc = xh - mean
    var = jnp.sum(xc * xc, axis=0, keepdims=True) / (n - 1.0)
    std = jnp.sqrt(var)
    std = jnp.where(std == 0.0, 1.0, std)
    deg = degp_ref[0] + degp_ref[1] + 1.0
    d = lax.rsqrt(deg)
    y0_ref[...] = (d * (xc / std))[None]
    deg_ref[...] = deg


def _mid_body(agg_ref, y_ref, deg_ref, out_ref):
    d2 = 1.0 / deg_ref[...]
    out_ref[...] = d2[None] * (agg_ref[...] + y_ref[...])


def _final_body(agg_ref, y_ref, deg_ref, out_ref):
    d = lax.rsqrt(deg_ref[...])
    out_ref[...] = d * (agg_ref[0] + y_ref[0])


_prep = pl.pallas_call(
    _prep_body,
    grid=(NC,),
    in_specs=[
        pl.BlockSpec((N, H), lambda c: (0, c)),
        pl.BlockSpec((NC, N, 1), lambda c: (0, 0, 0)),
    ],
    out_specs=[
        pl.BlockSpec((1, N, H), lambda c: (c, 0, 0)),
        pl.BlockSpec((N, 1), lambda c: (0, 0)),
    ],
    out_shape=[
        jax.ShapeDtypeStruct((NC, N, H), jnp.float32),
        jax.ShapeDtypeStruct((N, 1), jnp.float32),
    ],
)

_mid = pl.pallas_call(
    _mid_body,
    grid=(NC,),
    in_specs=[
        pl.BlockSpec((1, N, H), lambda c: (c, 0, 0)),
        pl.BlockSpec((1, N, H), lambda c: (c, 0, 0)),
        pl.BlockSpec((N, 1), lambda c: (0, 0)),
    ],
    out_specs=pl.BlockSpec((1, N, H), lambda c: (c, 0, 0)),
    out_shape=jax.ShapeDtypeStruct((NC, N, H), jnp.float32),
)

_final = pl.pallas_call(
    _final_body,
    grid=(NC,),
    in_specs=[
        pl.BlockSpec((1, N, H), lambda c: (c, 0, 0)),
        pl.BlockSpec((1, N, H), lambda c: (c, 0, 0)),
        pl.BlockSpec((N, 1), lambda c: (0, 0)),
    ],
    out_specs=pl.BlockSpec((N, H), lambda c: (0, c)),
    out_shape=jax.ShapeDtypeStruct((N, D), jnp.float32),
)


def kernel(x, edge_index):
    row = edge_index[0]
    col = edge_index[1]
    deg_parts = _deg_kernel(
        row, jnp.zeros((N,), jnp.float32), jnp.ones((CH_D,), jnp.float32)
    ).reshape(NC, N, 1)
    y0, deg = _prep(x, deg_parts)
    agg0 = _hop_kernel(y0, col, row)
    y1 = _mid(agg0, y0, deg)
    agg1 = _hop_kernel(y1, col, row)
    return _final(agg1, y1, deg)


# speedup vs baseline: 1.4576x; 1.1640x over previous
import functools

import jax
import jax.numpy as jnp
from jax import lax
from jax.experimental import pallas as pl
from jax.experimental.pallas import tpu as pltpu
from jax.experimental.pallas import tpu_sc as plsc

N = 10000
E = 160000
D = 256
H = 128
NC = 2
NS = 16
STRIPE = 624
TAIL = N - NS * STRIPE
TAIL_OFF = NS * STRIPE
EPT_HOP = E // NS
EPT_DEG = E // (NC * NS)
CH = 80
CH_D = 40

_MESH = plsc.VectorSubcoreMesh(
    core_axis_name="c", subcore_axis_name="s", num_cores=NC, num_subcores=NS
)


def _stripe_copy(src, dst, s):
    pltpu.sync_copy(
        src.at[pl.ds(s * STRIPE, STRIPE)], dst.at[pl.ds(s * STRIPE, STRIPE)]
    )
    @pl.when(s == 0)
    def _():
        pltpu.sync_copy(
            src.at[pl.ds(TAIL_OFF, TAIL)], dst.at[pl.ds(TAIL_OFF, TAIL)]
        )


@functools.partial(
    pl.kernel,
    out_type=jax.ShapeDtypeStruct((NC * N,), jnp.float32),
    mesh=_MESH,
    scratch_types=[
        pltpu.VMEM((EPT_DEG,), jnp.int32),
        pltpu.VMEM((CH_D,), jnp.float32),
        pltpu.VMEM((STRIPE,), jnp.float32),
        pltpu.VMEM_SHARED((N,), jnp.float32),
        pltpu.SemaphoreType.DMA,
    ],
)
def _deg_kernel(row_hbm, zeros_hbm, ones_hbm, out_hbm, idx_v, ones_v,
                stg_v, acc, sem_d):
    c = lax.axis_index("c")
    s = lax.axis_index("s")
    pltpu.sync_copy(zeros_hbm.at[pl.ds(0, STRIPE)], stg_v)
    pltpu.sync_copy(stg_v, acc.at[pl.ds(s * STRIPE, STRIPE)])
    @pl.when(s == 0)
    def _():
        pltpu.sync_copy(stg_v.at[pl.ds(0, TAIL)], acc.at[pl.ds(TAIL_OFF, TAIL)])
    pltpu.sync_copy(ones_hbm, ones_v)
    pltpu.sync_copy(row_hbm.at[pl.ds((c * NS + s) * EPT_DEG, EPT_DEG)], idx_v)
    plsc.subcore_barrier()

    def body(k, _):
        ia = idx_v.at[pl.ds(2 * k * CH_D, CH_D)]
        ib = idx_v.at[pl.ds((2 * k + 1) * CH_D, CH_D)]
        pltpu.async_copy(ones_v, acc.at[ia], sem_d, add=True)
        pltpu.async_copy(ones_v, acc.at[ib], sem_d, add=True)
        pltpu.make_async_copy(ones_v, acc.at[ia], sem_d).wait()
        pltpu.make_async_copy(ones_v, acc.at[ib], sem_d).wait()
        return 0

    nch_d = EPT_DEG // CH_D
    lax.fori_loop(0, nch_d // 2, body, 0)
    pltpu.sync_copy(
        ones_v, acc.at[idx_v.at[pl.ds((nch_d - 1) * CH_D, CH_D)]], add=True
    )
    plsc.subcore_barrier()
    pltpu.sync_copy(acc.at[pl.ds(s * STRIPE, STRIPE)], stg_v)
    pltpu.sync_copy(stg_v, out_hbm.at[pl.ds(c * N + s * STRIPE, STRIPE)])
    @pl.when(s == 0)
    def _():
        pltpu.sync_copy(acc.at[pl.ds(TAIL_OFF, TAIL)], stg_v.at[pl.ds(0, TAIL)])
        pltpu.sync_copy(
            stg_v.at[pl.ds(0, TAIL)], out_hbm.at[pl.ds(c * N + TAIL_OFF, TAIL)]
        )


@functools.partial(
    pl.kernel,
    out_type=jax.ShapeDtypeStruct((NC, N, H), jnp.float32),
    mesh=_MESH,
    scratch_types=[
        pltpu.VMEM((EPT_HOP,), jnp.int32),
        pltpu.VMEM((EPT_HOP,), jnp.int32),
        pltpu.VMEM((CH, H), jnp.float32),
        pltpu.VMEM((CH, H), jnp.float32),
        pltpu.VMEM((CH, H), jnp.float32),
        pltpu.VMEM_SHARED((N, H), jnp.float32),
        pltpu.SemaphoreType.DMA,
        pltpu.SemaphoreType.DMA,
        pltpu.SemaphoreType.DMA,
    ],
)
def _hop_kernel(y_hbm, col_hbm, row_hbm, out_hbm,
                col_v, row_v, buf_a, buf_b, buf_c, acc, sem_a, sem_b, sem_c):
    c = lax.axis_index("c")
    s = lax.axis_index("s")
    def zrow(r, _):
        for j in range(H // 16):
            buf_a[r, pl.ds(j * 16, 16)] = jnp.zeros((16,), jnp.float32)
        return 0

    lax.fori_loop(0, CH, zrow, 0)
    for j in range(STRIPE // CH):
        pltpu.sync_copy(buf_a, acc.at[pl.ds(s * STRIPE + j * CH, CH)])
    pltpu.sync_copy(
        buf_a.at[pl.ds(0, STRIPE - (STRIPE // CH) * CH)],
        acc.at[pl.ds(s * STRIPE + (STRIPE // CH) * CH,
                     STRIPE - (STRIPE // CH) * CH)],
    )
    @pl.when(s == 0)
    def _():
        pltpu.sync_copy(buf_a.at[pl.ds(0, TAIL)], acc.at[pl.ds(TAIL_OFF, TAIL)])
    y_half = y_hbm.at[c]
    nch = EPT_HOP // CH
    pltpu.sync_copy(col_hbm.at[pl.ds(s * EPT_HOP, EPT_HOP)], col_v)
    pltpu.sync_copy(row_hbm.at[pl.ds(s * EPT_HOP, EPT_HOP)], row_v)
    plsc.subcore_barrier()

    def cidx(k):
        return col_v.at[pl.ds(k * CH, CH)]

    def ridx(k):
        return row_v.at[pl.ds(k * CH, CH)]

    pltpu.async_copy(y_half.at[cidx(0)], buf_a, sem_a)
    pltpu.async_copy(y_half.at[cidx(1)], buf_b, sem_b)

    def trio(j, _):
        k = 3 * j
        pltpu.async_copy(y_half.at[cidx(k + 2)], buf_c, sem_c)
        pltpu.make_async_copy(y_half.at[cidx(k)], buf_a, sem_a).wait()
        pltpu.sync_copy(buf_a, acc.at[ridx(k)], add=True)
        pltpu.async_copy(y_half.at[cidx(k + 3)], buf_a, sem_a)
        pltpu.make_async_copy(y_half.at[cidx(k + 1)], buf_b, sem_b).wait()
        pltpu.sync_copy(buf_b, acc.at[ridx(k + 1)], add=True)
        pltpu.async_copy(y_half.at[cidx(k + 4)], buf_b, sem_b)
        pltpu.make_async_copy(y_half.at[cidx(k + 2)], buf_c, sem_c).wait()
        pltpu.sync_copy(buf_c, acc.at[ridx(k + 2)], add=True)
        return 0

    lax.fori_loop(0, (nch - 2) // 3, trio, 0)
    pltpu.make_async_copy(y_half.at[cidx(nch - 2)], buf_a, sem_a).wait()
    pltpu.sync_copy(buf_a, acc.at[ridx(nch - 2)], add=True)
    pltpu.make_async_copy(y_half.at[cidx(nch - 1)], buf_b, sem_b).wait()
    pltpu.sync_copy(buf_b, acc.at[ridx(nch - 1)], add=True)
    plsc.subcore_barrier()
    _stripe_copy(acc, out_hbm.at[c], s)


def _prep_body(x_ref, degp_ref, y0_ref, deg_ref):
    xh = x_ref[...]
    n = jnp.float32(N)
    mean = jnp.sum(xh, axis=0, keepdims=True) / n
    xc = xh - mean
    var = jnp.sum(xc * xc, axis=0, keepdims=True) / (n - 1.0)
    std = jnp.sqrt(var)
    std = jnp.where(std == 0.0, 1.0, std)
    deg = degp_ref[0] + degp_ref[1] + 1.0
    d = lax.rsqrt(deg)
    y0_ref[...] = (d * (xc / std))[None]
    deg_ref[...] = deg


def _mid_body(agg_ref, y_ref, deg_ref, out_ref):
    d2 = 1.0 / deg_ref[...]
    out_ref[...] = d2[None] * (agg_ref[...] + y_ref[...])


def _final_body(agg_ref, y_ref, deg_ref, out_ref):
    d = lax.rsqrt(deg_ref[...])
    out_ref[...] = d * (agg_ref[0] + y_ref[0])


_prep = pl.pallas_call(
    _prep_body,
    grid=(NC,),
    in_specs=[
        pl.BlockSpec((N, H), lambda c: (0, c)),
        pl.BlockSpec((NC, N, 1), lambda c: (0, 0, 0)),
    ],
    out_specs=[
        pl.BlockSpec((1, N, H), lambda c: (c, 0, 0)),
        pl.BlockSpec((N, 1), lambda c: (0, 0)),
    ],
    out_shape=[
        jax.ShapeDtypeStruct((NC, N, H), jnp.float32),
        jax.ShapeDtypeStruct((N, 1), jnp.float32),
    ],
)

_mid = pl.pallas_call(
    _mid_body,
    grid=(NC,),
    in_specs=[
        pl.BlockSpec((1, N, H), lambda c: (c, 0, 0)),
        pl.BlockSpec((1, N, H), lambda c: (c, 0, 0)),
        pl.BlockSpec((N, 1), lambda c: (0, 0)),
    ],
    out_specs=pl.BlockSpec((1, N, H), lambda c: (c, 0, 0)),
    out_shape=jax.ShapeDtypeStruct((NC, N, H), jnp.float32),
)

_final = pl.pallas_call(
    _final_body,
    grid=(NC,),
    in_specs=[
        pl.BlockSpec((1, N, H), lambda c: (c, 0, 0)),
        pl.BlockSpec((1, N, H), lambda c: (c, 0, 0)),
        pl.BlockSpec((N, 1), lambda c: (0, 0)),
    ],
    out_specs=pl.BlockSpec((N, H), lambda c: (0, c)),
    out_shape=jax.ShapeDtypeStruct((N, D), jnp.float32),
)


def kernel(x, edge_index):
    row = edge_index[0]
    col = edge_index[1]
    deg_parts = _deg_kernel(
        row, jnp.zeros((N,), jnp.float32), jnp.ones((CH_D,), jnp.float32)
    ).reshape(NC, N, 1)
    y0, deg = _prep(x, deg_parts)
    agg0 = _hop_kernel(y0, col, row)
    y1 = _mid(agg0, y0, deg)
    agg1 = _hop_kernel(y1, col, row)
    return _final(agg1, y1, deg)
